# baseline (speedup 1.0000x reference)
import functools

import jax
import jax.numpy as jnp
from jax import lax
from jax.experimental import pallas as pl
from jax.experimental.pallas import tpu as pltpu
from jax.experimental.pallas import tpu_sc as plsc

NUM_CORES = 2
NUM_SUBCORES = 16
NUM_WORKERS = NUM_CORES * NUM_SUBCORES
NBUF = 2


@functools.partial(jax.jit, static_argnums=(0, 1, 2, 3))
def _embedding_lookup(b, l, v, d, x, table):
    b_per_w = b // NUM_WORKERS
    n_iters = b_per_w // NBUF
    mesh = plsc.VectorSubcoreMesh(core_axis_name="c", subcore_axis_name="s")

    @functools.partial(
        pl.kernel,
        out_type=jax.ShapeDtypeStruct((b, l, d), jnp.float32),
        mesh=mesh,
        scratch_types=[
            pltpu.VMEM((b_per_w, l), jnp.int32),
            [pltpu.VMEM((l, d), jnp.float32)] * NBUF,
            [pltpu.SemaphoreType.DMA] * NBUF,
            [pltpu.SemaphoreType.DMA] * NBUF,
        ],
        compiler_params=pltpu.CompilerParams(use_tc_tiling_on_sc=False),
    )
    def lookup(x_hbm, table_hbm, out_hbm, idx_v, rows, g, s):
        wid = lax.axis_index("s") * NUM_CORES + lax.axis_index("c")
        base = wid * b_per_w
        pltpu.sync_copy(x_hbm.at[pl.ds(base, b_per_w)], idx_v)

        def gather_desc(c, k):
            return pltpu.make_async_copy(
                table_hbm.at[idx_v.at[c]], rows[k], g[k]
            )

        def scatter_desc(c, k):
            return pltpu.make_async_copy(rows[k], out_hbm.at[base + c], s[k])

        for k in range(NBUF):
            gather_desc(k, k).start()

        def body(i, carry):
            c0 = i * NBUF
            for k in range(NBUF):
                gather_desc(c0 + k, k).wait()
                scatter_desc(c0 + k, k).start()
            for k in range(NBUF):
                scatter_desc(c0 + k, k).wait()
                gather_desc(c0 + NBUF + k, k).start()
            return carry

        lax.fori_loop(0, n_iters - 1, body, 0)

        c0 = (n_iters - 1) * NBUF
        for k in range(NBUF):
            gather_desc(c0 + k, k).wait()
            scatter_desc(c0 + k, k).start()
        for k in range(NBUF):
            scatter_desc(c0 + k, k).wait()

    return lookup(x, table)


def kernel(x, token_embedding):
    b, l = x.shape
    v, d = token_embedding.shape
    return _embedding_lookup(b, l, v, d, x.astype(jnp.int32), token_embedding)
